# baseline (speedup 1.0000x reference)
import functools

import jax
import jax.numpy as jnp
from jax import lax
from jax.experimental import pallas as pl
from jax.experimental.pallas import tpu as pltpu
from jax.experimental.pallas import tpu_sc as plsc

NPROT = 10000
NHE = 2000
EDGES = 320000
D = 128
LANES = 16
NCORES = 2
NSUB = 16
NWORK = NCORES * NSUB
CHUNK = 96


GRP = 8


def _scatter_pass_body(nseg, ntab, nchunk, chunk):
  def _stripe_counts(n):
    ceil_div = lambda a, b: -(-a // b)
    main = ceil_div(ceil_div(n, NSUB), 8) * 8
    last = n - (NSUB - 1) * main
    assert last > 0 and last % 8 == 0 and main % 8 == 0
    return main, last

  rpt, rpt_last = _stripe_counts(nseg)
  tpt, tpt_last = _stripe_counts(ntab)
  ngrp = nchunk // GRP

  def _emit_striped(sid, counts, per, fn):
    main, lastc = counts

    def emit(cnt):
      off = 0
      while off < cnt:
        c = min(per, cnt - off)
        fn(sid * main + off, c)
        off += c

    @pl.when(sid < NSUB - 1)
    def _():
      emit(main)

    @pl.when(sid == NSUB - 1)
    def _():
      emit(lastc)

  def body(table_hbm, gidx_hbm, sidx_hbm, w_hbm, out_hbm,
           gb0, gb1, sb0, sb1, wb0, wb1, grows0_v, grows1_v, srows0_v,
           table_sh, acc_sh,
           semr0, semr1, semi0, semi1, sems0, sems1):
    cid = lax.axis_index("c")
    sid = lax.axis_index("s")
    wid = sid * NCORES + cid

    _emit_striped(
        sid, (tpt, tpt_last), 512,
        lambda b, c: pltpu.sync_copy(table_hbm.at[pl.ds(b, c)],
                                     table_sh.at[pl.ds(b, c)]))

    gbufs = (gb0, gb1)
    sbufs = (sb0, sb1)
    wbufs = (wb0, wb1)
    grows = (grows0_v, grows1_v)
    srows = (srows0_v, srows0_v)
    rsems = (semr0, semr1)
    isems = (semi0, semi1)
    ssems = (sems0, sems1)

    def prefetch_group(g, b):
      pltpu.async_copy(gidx_hbm.at[wid, g], gbufs[b], isems[b])
      pltpu.async_copy(sidx_hbm.at[wid, g], sbufs[b], isems[b])
      pltpu.async_copy(w_hbm.at[wid, g], wbufs[b], isems[b])

    def wait_group(g, b):
      pltpu.make_async_copy(gidx_hbm.at[wid, g], gbufs[b], isems[b]).wait()
      pltpu.make_async_copy(sidx_hbm.at[wid, g], sbufs[b], isems[b]).wait()
      pltpu.make_async_copy(w_hbm.at[wid, g], wbufs[b], isems[b]).wait()

    def start_gather(idx_row, rb):
      pltpu.async_copy(table_sh.at[idx_row], grows[rb], rsems[rb])

    def wait_gather(idx_row, rb):
      pltpu.make_async_copy(table_sh.at[idx_row], grows[rb],
                            rsems[rb]).wait()

    def scale(k, bg, rb):
      gbuf = grows[rb]
      sbuf = srows[rb]
      wrow = wbufs[bg]

      def expand_body(r, c2):
        for q in range(D // (2 * LANES)):
          v = gbuf[r, pl.ds(q * LANES, LANES)]
          sbuf[r, pl.ds(q * 2 * LANES, LANES)] = plsc.bitcast(
              v << 16, jnp.float32)
          sbuf[r, pl.ds(q * 2 * LANES + LANES, LANES)] = plsc.bitcast(
              v & jnp.int32(-65536), jnp.float32)
        return c2

      lax.fori_loop(0, chunk, expand_body, 0)

      def grp_body(g16, c2):
        wvec = wrow[k, pl.ds(g16 * LANES, LANES)]
        for i in range(LANES):
          w = wvec[i]
          r = g16 * LANES + i
          for j in range(D // LANES):
            sl = pl.ds(j * LANES, LANES)
            sbuf[r, sl] = sbuf[r, sl] * w
        return c2

      lax.fori_loop(0, chunk // LANES, grp_body, 0)

    def start_scatter(k, bg, rb):
      pltpu.async_copy(srows[rb], acc_sh.at[sbufs[bg].at[k]], ssems[rb],
                       add=True)

    def wait_scatter(k, bg, rb):
      pltpu.make_async_copy(srows[rb], acc_sh.at[sbufs[bg].at[k]],
                            ssems[rb]).wait()

    zero = jnp.zeros((LANES,), jnp.float32)

    def zrow(i, carry):
      for j in range(D // LANES):
        srows0_v[i, pl.ds(j * LANES, LANES)] = zero
      return carry

    lax.fori_loop(0, chunk, zrow, 0)

    _emit_striped(
        sid, (rpt, rpt_last), chunk,
        lambda b, c: pltpu.sync_copy(srows0_v.at[pl.ds(0, c)],
                                     acc_sh.at[pl.ds(b, c)]))
    plsc.subcore_barrier()

    prefetch_group(0, 0)
    wait_group(0, 0)
    start_gather(gb0.at[0], 0)

    def pair_body(go, carry):
      for g2 in range(2):
        bg = g2
        g = go * 2 + g2

        for k in range(GRP):
          rb = k % 2

          wait_gather(gbufs[bg].at[k], rb)

          if k > 0:
            wait_scatter(k - 1, bg, 1 - rb)
          else:

            @pl.when(g > 0)
            def _():
              wait_scatter(GRP - 1, 1 - bg, 1 - rb)

          if k == 0:
            @pl.when(g + 1 < ngrp)
            def _():
              prefetch_group(g + 1, 1 - bg)

          if k + 1 < GRP:
            start_gather(gbufs[bg].at[k + 1], 1 - rb)
          else:

            @pl.when(g + 1 < ngrp)
            def _():
              wait_group(g + 1, 1 - bg)
              start_gather(gbufs[1 - bg].at[0], 1 - rb)

          scale(k, bg, rb)
          start_scatter(k, bg, rb)
      return carry

    lax.fori_loop(0, ngrp // 2, pair_body, 0)
    wait_scatter(GRP - 1, (ngrp - 1) % 2, (GRP - 1) % 2)
    plsc.subcore_barrier()

    _emit_striped(
        sid, (rpt, rpt_last), 512,
        lambda b, c: pltpu.sync_copy(acc_sh.at[pl.ds(b, c)],
                                     out_hbm.at[cid, pl.ds(b, c)]))

  return body


@functools.partial(jax.jit, static_argnames=("nseg", "chunk"))
def _run_scatter_pass(table, gidx, sidx, w, nseg, chunk=CHUNK):
  ntab = table.shape[0]
  epw = EDGES // NWORK
  nchunk = -(-epw // (2 * GRP * chunk)) * (2 * GRP)
  ngrp = nchunk // GRP
  epad = NWORK * nchunk * chunk
  pad = epad - EDGES
  gidx_p = jnp.concatenate([gidx, jnp.zeros((pad,), jnp.int32)]).reshape(
      NWORK, ngrp, GRP, chunk)
  sidx_p = jnp.concatenate([sidx, jnp.zeros((pad,), jnp.int32)]).reshape(
      NWORK, ngrp, GRP, chunk)
  w_p = jnp.concatenate([w, jnp.zeros((pad,), jnp.float32)]).reshape(
      NWORK, ngrp, GRP, chunk)

  kfn = pl.kernel(
      _scatter_pass_body(nseg, ntab, nchunk, chunk),
      out_type=jax.ShapeDtypeStruct((NCORES, nseg, D), jnp.float32),
      mesh=plsc.VectorSubcoreMesh(core_axis_name="c", subcore_axis_name="s",
                                  num_cores=NCORES, num_subcores=NSUB),
      compiler_params=pltpu.CompilerParams(needs_layout_passes=False,
                                           use_tc_tiling_on_sc=False),
      scratch_types=[
          pltpu.VMEM((GRP, chunk), jnp.int32),
          pltpu.VMEM((GRP, chunk), jnp.int32),
          pltpu.VMEM((GRP, chunk), jnp.int32),
          pltpu.VMEM((GRP, chunk), jnp.int32),
          pltpu.VMEM((GRP, chunk), jnp.float32),
          pltpu.VMEM((GRP, chunk), jnp.float32),
          pltpu.VMEM((chunk, D // 2), jnp.int32),
          pltpu.VMEM((chunk, D // 2), jnp.int32),
          pltpu.VMEM((chunk, D), jnp.float32),
          pltpu.VMEM_SHARED((ntab, D // 2), jnp.int32),
          pltpu.VMEM_SHARED((nseg, D), jnp.float32),
          pltpu.SemaphoreType.DMA,
          pltpu.SemaphoreType.DMA,
          pltpu.SemaphoreType.DMA,
          pltpu.SemaphoreType.DMA,
          pltpu.SemaphoreType.DMA,
          pltpu.SemaphoreType.DMA,
      ],
  )
  return kfn(table, gidx_p, sidx_p, w_p)


def _attn_body(hp_ref, w1_ref, b1_ref, w2_ref, b2_ref, wf_ref, out_ref):
  h = hp_ref[0] + hp_ref[1]
  hh = jnp.maximum(
      jnp.dot(h, w1_ref[...], preferred_element_type=jnp.float32) + b1_ref[...],
      0.0)
  a = jnp.dot(hh, w2_ref[...], preferred_element_type=jnp.float32) + b2_ref[...]
  a = 1.0 / (1.0 + jnp.exp(-a))
  s = jnp.dot(a, wf_ref[...], preferred_element_type=jnp.float32)
  out_ref[...] = h * s


def _final_body(feat_ref, cp_ref, ws_ref, bs_ref, wc_ref, bc_ref,
                w1a_ref, w1b_ref, b1_ref, w2_ref, b2_ref, w3_ref, b3_ref,
                out_ref):
  f = feat_ref[...]
  c = cp_ref[0] + cp_ref[1]
  sf = jnp.dot(f, ws_ref[...], preferred_element_type=jnp.float32) + bs_ref[...]
  ct = jnp.dot(c, wc_ref[...], preferred_element_type=jnp.float32) + bc_ref[...]
  h1 = jnp.maximum(
      jnp.dot(sf, w1a_ref[...], preferred_element_type=jnp.float32)
      + jnp.dot(ct, w1b_ref[...], preferred_element_type=jnp.float32)
      + b1_ref[...], 0.0)
  h2 = jnp.maximum(
      jnp.dot(h1, w2_ref[...], preferred_element_type=jnp.float32)
      + b2_ref[...], 0.0)
  t = jnp.dot(h2, w3_ref[...], preferred_element_type=jnp.float32) + b3_ref[...]
  w0 = 1.0 / (1.0 + jnp.exp(-t))
  fused = sf * w0 + ct * (1.0 - w0) + f
  out_ref[...] = jnp.maximum(fused, 0.0)


def _pack_table(x):
  b = x.astype(jnp.bfloat16).reshape(-1, D // 32, 2, LANES)
  b = b.transpose(0, 1, 3, 2)
  return jax.lax.bitcast_convert_type(b, jnp.int32).reshape(-1, D // 2)


def kernel(feat, inc_src, inc_dst, edge_weight,
           Ws, bs, Wc, bc, Wh1, bh1, Wh2, bh2, Wf,
           Wfa1, bfa1, Wfa2, bfa2, Wfa3, bfa3):
  n_heads, head_dim, _ = Wh1.shape

  hp = _run_scatter_pass(_pack_table(feat), inc_src, inc_dst, edge_weight, NHE)

  w1 = Wh1.reshape(n_heads * head_dim, D).T
  b1 = bh1.reshape(1, n_heads * head_dim)
  w2 = jax.scipy.linalg.block_diag(*[Wh2[i].T for i in range(n_heads)])
  b2 = bh2.reshape(1, n_heads)
  wf = Wf.T

  hew = pl.pallas_call(
      _attn_body,
      out_shape=jax.ShapeDtypeStruct((NHE, D), jnp.float32),
  )(hp, w1, b1, w2, b2, wf)

  cp = _run_scatter_pass(_pack_table(hew), inc_dst, inc_src, edge_weight,
                         NPROT)

  nblk = max(1, NPROT // 2000)
  blk = NPROT // nblk
  ws_t = Ws.T
  wc_t = Wc.T
  w1a = Wfa1[:, :D].T
  w1b = Wfa1[:, D:].T
  b1f = bfa1.reshape(1, -1)
  w2f = Wfa2.T
  b2f = bfa2.reshape(1, -1)
  w3f = (Wfa3[0] - Wfa3[1]).reshape(-1, 1)
  b3f = (bfa3[0] - bfa3[1]).reshape(1, 1)

  full = lambda shape: pl.BlockSpec(shape, lambda i: (0,) * len(shape))
  out = pl.pallas_call(
      _final_body,
      grid=(nblk,),
      in_specs=[
          pl.BlockSpec((blk, D), lambda i: (i, 0)),
          pl.BlockSpec((NCORES, blk, D), lambda i: (0, i, 0)),
          full((D, D)), full((1, D)), full((D, D)), full((1, D)),
          full((D, D)), full((D, D)), full((1, D)),
          full((D, 64)), full((1, 64)), full((64, 1)), full((1, 1)),
      ],
      out_specs=pl.BlockSpec((blk, D), lambda i: (i, 0)),
      out_shape=jax.ShapeDtypeStruct((NPROT, D), jnp.float32),
  )(feat, cp, ws_t, bs.reshape(1, -1), wc_t, bc.reshape(1, -1),
    w1a, w1b, b1f, w2f, b2f, w3f, b3f)
  return out


# speedup vs baseline: 2.0831x; 2.0831x over previous
import functools

import jax
import jax.numpy as jnp
from jax import lax
from jax.experimental import pallas as pl
from jax.experimental.pallas import tpu as pltpu
from jax.experimental.pallas import tpu_sc as plsc

NPROT = 10000
NHE = 2000
EDGES = 320000
D = 128
LANES = 16
NCORES = 2
NSUB = 16
NWORK = NCORES * NSUB
CHUNK = 112


GRP = 8


def _scatter_pass_body(nseg, ntab, nchunk, chunk):
  def _stripe_counts(n):
    ceil_div = lambda a, b: -(-a // b)
    main = ceil_div(ceil_div(n, NSUB), 8) * 8
    last = n - (NSUB - 1) * main
    assert last > 0 and last % 8 == 0 and main % 8 == 0
    return main, last

  rpt, rpt_last = _stripe_counts(nseg)
  tpt, tpt_last = _stripe_counts(ntab)
  ngrp = nchunk // GRP

  def _emit_striped(sid, counts, per, fn):
    main, lastc = counts

    def emit(cnt):
      off = 0
      while off < cnt:
        c = min(per, cnt - off)
        fn(sid * main + off, c)
        off += c

    @pl.when(sid < NSUB - 1)
    def _():
      emit(main)

    @pl.when(sid == NSUB - 1)
    def _():
      emit(lastc)

  def body(table_hbm, gidx_hbm, sidx_hbm, w_hbm, out_hbm,
           gb0, gb1, sb0, sb1, wb0, wb1, rows0_v, rows1_v, table_sh, acc_sh,
           semr0, semr1, semi0, semi1, sems0, sems1):
    cid = lax.axis_index("c")
    sid = lax.axis_index("s")
    wid = sid * NCORES + cid

    _emit_striped(
        sid, (tpt, tpt_last), 512,
        lambda b, c: pltpu.sync_copy(table_hbm.at[pl.ds(b, c)],
                                     table_sh.at[pl.ds(b, c)]))

    gbufs = (gb0, gb1)
    sbufs = (sb0, sb1)
    wbufs = (wb0, wb1)
    rbufs = (rows0_v, rows1_v)
    rsems = (semr0, semr1)
    isems = (semi0, semi1)
    ssems = (sems0, sems1)

    def prefetch_group(g, b):
      pltpu.async_copy(gidx_hbm.at[wid, g], gbufs[b], isems[b])
      pltpu.async_copy(sidx_hbm.at[wid, g], sbufs[b], isems[b])
      pltpu.async_copy(w_hbm.at[wid, g], wbufs[b], isems[b])

    def wait_group(g, b):
      pltpu.make_async_copy(gidx_hbm.at[wid, g], gbufs[b], isems[b]).wait()
      pltpu.make_async_copy(sidx_hbm.at[wid, g], sbufs[b], isems[b]).wait()
      pltpu.make_async_copy(w_hbm.at[wid, g], wbufs[b], isems[b]).wait()

    def start_gather(idx_row, rb):
      pltpu.async_copy(table_sh.at[idx_row], rbufs[rb], rsems[rb])

    def wait_gather(idx_row, rb):
      pltpu.make_async_copy(table_sh.at[idx_row], rbufs[rb],
                            rsems[rb]).wait()

    def scale(k, bg, rb):
      buf = rbufs[rb]
      wrow = wbufs[bg]

      def grp_body(g16, c2):
        wvec = wrow[k, pl.ds(g16 * LANES, LANES)]
        for i in range(LANES):
          w = wvec[i]
          r = g16 * LANES + i
          for j in range(D // LANES):
            sl = pl.ds(j * LANES, LANES)
            buf[r, sl] = buf[r, sl] * w
        return c2

      lax.fori_loop(0, chunk // LANES, grp_body, 0)

    def start_scatter(k, bg, rb):
      pltpu.async_copy(rbufs[rb], acc_sh.at[sbufs[bg].at[k]], ssems[rb],
                       add=True)

    def wait_scatter(k, bg, rb):
      pltpu.make_async_copy(rbufs[rb], acc_sh.at[sbufs[bg].at[k]],
                            ssems[rb]).wait()

    zero = jnp.zeros((LANES,), jnp.float32)

    def zrow(i, carry):
      for j in range(D // LANES):
        rows0_v[i, pl.ds(j * LANES, LANES)] = zero
      return carry

    lax.fori_loop(0, chunk, zrow, 0)

    _emit_striped(
        sid, (rpt, rpt_last), chunk,
        lambda b, c: pltpu.sync_copy(rows0_v.at[pl.ds(0, c)],
                                     acc_sh.at[pl.ds(b, c)]))
    plsc.subcore_barrier()

    prefetch_group(0, 0)
    wait_group(0, 0)
    start_gather(gb0.at[0], 0)

    def pair_body(go, carry):
      for g2 in range(2):
        bg = g2
        g = go * 2 + g2

        for k in range(GRP):
          rb = k % 2

          wait_gather(gbufs[bg].at[k], rb)

          if k > 0:
            wait_scatter(k - 1, bg, 1 - rb)
          else:

            @pl.when(g > 0)
            def _():
              wait_scatter(GRP - 1, 1 - bg, 1 - rb)

          if k == 0:
            @pl.when(g + 1 < ngrp)
            def _():
              prefetch_group(g + 1, 1 - bg)

          if k + 1 < GRP:
            start_gather(gbufs[bg].at[k + 1], 1 - rb)
          else:

            @pl.when(g + 1 < ngrp)
            def _():
              wait_group(g + 1, 1 - bg)
              start_gather(gbufs[1 - bg].at[0], 1 - rb)

          scale(k, bg, rb)
          start_scatter(k, bg, rb)
      return carry

    lax.fori_loop(0, ngrp // 2, pair_body, 0)
    wait_scatter(GRP - 1, (ngrp - 1) % 2, (GRP - 1) % 2)
    plsc.subcore_barrier()

    _emit_striped(
        sid, (rpt, rpt_last), 512,
        lambda b, c: pltpu.sync_copy(acc_sh.at[pl.ds(b, c)],
                                     out_hbm.at[cid, pl.ds(b, c)]))

  return body


@functools.partial(jax.jit, static_argnames=("nseg", "chunk"))
def _run_scatter_pass(table, gidx, sidx, w, nseg, chunk=CHUNK):
  ntab = table.shape[0]
  epw = EDGES // NWORK
  nchunk = -(-epw // (2 * GRP * chunk)) * (2 * GRP)
  ngrp = nchunk // GRP
  epad = NWORK * nchunk * chunk
  pad = epad - EDGES
  gidx_p = jnp.concatenate([gidx, jnp.zeros((pad,), jnp.int32)]).reshape(
      NWORK, ngrp, GRP, chunk)
  sidx_p = jnp.concatenate([sidx, jnp.zeros((pad,), jnp.int32)]).reshape(
      NWORK, ngrp, GRP, chunk)
  w_p = jnp.concatenate([w, jnp.zeros((pad,), jnp.float32)]).reshape(
      NWORK, ngrp, GRP, chunk)

  kfn = pl.kernel(
      _scatter_pass_body(nseg, ntab, nchunk, chunk),
      out_type=jax.ShapeDtypeStruct((NCORES, nseg, D), jnp.float32),
      mesh=plsc.VectorSubcoreMesh(core_axis_name="c", subcore_axis_name="s",
                                  num_cores=NCORES, num_subcores=NSUB),
      scratch_types=[
          pltpu.VMEM((GRP, chunk), jnp.int32),
          pltpu.VMEM((GRP, chunk), jnp.int32),
          pltpu.VMEM((GRP, chunk), jnp.int32),
          pltpu.VMEM((GRP, chunk), jnp.int32),
          pltpu.VMEM((GRP, chunk), jnp.float32),
          pltpu.VMEM((GRP, chunk), jnp.float32),
          pltpu.VMEM((chunk, D), jnp.float32),
          pltpu.VMEM((chunk, D), jnp.float32),
          pltpu.VMEM_SHARED((ntab, D), jnp.float32),
          pltpu.VMEM_SHARED((nseg, D), jnp.float32),
          pltpu.SemaphoreType.DMA,
          pltpu.SemaphoreType.DMA,
          pltpu.SemaphoreType.DMA,
          pltpu.SemaphoreType.DMA,
          pltpu.SemaphoreType.DMA,
          pltpu.SemaphoreType.DMA,
      ],
  )
  return kfn(table, gidx_p, sidx_p, w_p)


def _attn_body(hp_ref, w1_ref, b1_ref, w2_ref, b2_ref, wf_ref, out_ref):
  h = hp_ref[0] + hp_ref[1]
  hh = jnp.maximum(
      jnp.dot(h, w1_ref[...], preferred_element_type=jnp.float32) + b1_ref[...],
      0.0)
  a = jnp.dot(hh, w2_ref[...], preferred_element_type=jnp.float32) + b2_ref[...]
  a = 1.0 / (1.0 + jnp.exp(-a))
  s = jnp.dot(a, wf_ref[...], preferred_element_type=jnp.float32)
  out_ref[...] = h * s


def _final_body(feat_ref, cp_ref, ws_ref, bs_ref, wc_ref, bc_ref,
                w1a_ref, w1b_ref, b1_ref, w2_ref, b2_ref, w3_ref, b3_ref,
                out_ref):
  f = feat_ref[...]
  c = cp_ref[0] + cp_ref[1]
  sf = jnp.dot(f, ws_ref[...], preferred_element_type=jnp.float32) + bs_ref[...]
  ct = jnp.dot(c, wc_ref[...], preferred_element_type=jnp.float32) + bc_ref[...]
  h1 = jnp.maximum(
      jnp.dot(sf, w1a_ref[...], preferred_element_type=jnp.float32)
      + jnp.dot(ct, w1b_ref[...], preferred_element_type=jnp.float32)
      + b1_ref[...], 0.0)
  h2 = jnp.maximum(
      jnp.dot(h1, w2_ref[...], preferred_element_type=jnp.float32)
      + b2_ref[...], 0.0)
  t = jnp.dot(h2, w3_ref[...], preferred_element_type=jnp.float32) + b3_ref[...]
  w0 = 1.0 / (1.0 + jnp.exp(-t))
  fused = sf * w0 + ct * (1.0 - w0) + f
  out_ref[...] = jnp.maximum(fused, 0.0)


def kernel(feat, inc_src, inc_dst, edge_weight,
           Ws, bs, Wc, bc, Wh1, bh1, Wh2, bh2, Wf,
           Wfa1, bfa1, Wfa2, bfa2, Wfa3, bfa3):
  n_heads, head_dim, _ = Wh1.shape

  hp = _run_scatter_pass(feat, inc_src, inc_dst, edge_weight, NHE)

  w1 = Wh1.reshape(n_heads * head_dim, D).T
  b1 = bh1.reshape(1, n_heads * head_dim)
  w2 = jax.scipy.linalg.block_diag(*[Wh2[i].T for i in range(n_heads)])
  b2 = bh2.reshape(1, n_heads)
  wf = Wf.T

  hew = pl.pallas_call(
      _attn_body,
      out_shape=jax.ShapeDtypeStruct((NHE, D), jnp.float32),
  )(hp, w1, b1, w2, b2, wf)

  cp = _run_scatter_pass(hew, inc_dst, inc_src, edge_weight, NPROT)

  nblk = max(1, NPROT // 2000)
  blk = NPROT // nblk
  ws_t = Ws.T
  wc_t = Wc.T
  w1a = Wfa1[:, :D].T
  w1b = Wfa1[:, D:].T
  b1f = bfa1.reshape(1, -1)
  w2f = Wfa2.T
  b2f = bfa2.reshape(1, -1)
  w3f = (Wfa3[0] - Wfa3[1]).reshape(-1, 1)
  b3f = (bfa3[0] - bfa3[1]).reshape(1, 1)

  full = lambda shape: pl.BlockSpec(shape, lambda i: (0,) * len(shape))
  out = pl.pallas_call(
      _final_body,
      grid=(nblk,),
      in_specs=[
          pl.BlockSpec((blk, D), lambda i: (i, 0)),
          pl.BlockSpec((NCORES, blk, D), lambda i: (0, i, 0)),
          full((D, D)), full((1, D)), full((D, D)), full((1, D)),
          full((D, D)), full((D, D)), full((1, D)),
          full((D, 64)), full((1, 64)), full((64, 1)), full((1, 1)),
      ],
      out_specs=pl.BlockSpec((blk, D), lambda i: (i, 0)),
      out_shape=jax.ShapeDtypeStruct((NPROT, D), jnp.float32),
  )(feat, cp, ws_t, bs.reshape(1, -1), wc_t, bc.reshape(1, -1),
    w1a, w1b, b1f, w2f, b2f, w3f, b3f)
  return out
